# baseline (speedup 1.0000x reference)
import functools

import jax
import jax.numpy as jnp
from jax.experimental import pallas as pl
from jax.experimental.pallas import tpu as pltpu

_COMMIT = 0.25


def _vq_pair_kernel(x_ref, wd_ref, wg_ref, e2_ref, q_ref, idx_ref, csum_ref,
                    *, G, M, K, d):
    BLK = x_ref.shape[0]
    P = G // 2
    step = pl.program_id(0)

    @pl.when(step == 0)
    def _init():
        csum_ref[:, :] = jnp.zeros((1, 1), jnp.float32)

    acc = jnp.float32(0.0)
    dn = (((1,), (0,)), ((), ()))
    iota = jax.lax.broadcasted_iota(jnp.int32, (BLK, K), 1)
    for c in range(P):
        rp = x_ref[:, 2 * d * c:2 * d * (c + 1)]
        rec = jnp.zeros_like(rp)
        for m in range(M):
            s = rp * rp
            p = s
            for k in range(1, 8):
                p = p + jnp.roll(s, -8 * k, axis=1)
            t = p + jnp.roll(p, -4, axis=1)
            t = t + jnp.roll(t, -2, axis=1)
            t = t + jnp.roll(t, -1, axis=1)
            r2a = t[:, 0:1]
            r2b = t[:, d:d + 1]
            dots = jax.lax.dot_general(
                rp, wd_ref[c, m], dn,
                preferred_element_type=jnp.float32)
            e2row = e2_ref[c * M + m:c * M + m + 1, :]
            da = (r2a - 2.0 * dots[:, :K]) + e2row[:, :K]
            db = (r2b - 2.0 * dots[:, K:]) + e2row[:, K:]
            mina = jnp.min(da, axis=1, keepdims=True)
            ia = jnp.min(jnp.where(da == mina, iota, K), axis=1,
                         keepdims=True)
            minb = jnp.min(db, axis=1, keepdims=True)
            ib = jnp.min(jnp.where(db == minb, iota, K), axis=1,
                         keepdims=True)
            oh_a = (iota == ia).astype(jnp.float32)
            oh_b = (iota == ib).astype(jnp.float32)
            q3a = jax.lax.dot_general(
                oh_a, wg_ref[2 * c, m], dn,
                preferred_element_type=jnp.float32)
            q3b = jax.lax.dot_general(
                oh_b, wg_ref[2 * c + 1, m], dn,
                preferred_element_type=jnp.float32)
            qa = (q3a[:, 0:d] + q3a[:, d:2 * d]) + q3a[:, 2 * d:3 * d]
            qb = (q3b[:, 0:d] + q3b[:, d:2 * d]) + q3b[:, 2 * d:3 * d]
            qp = jnp.concatenate([qa, qb], axis=1)
            acc += jnp.sum((qp - rp) ** 2)
            qst = rp + (qp - rp)
            rec = rec + qst
            rp = rp - qst
            ca = (2 * c) * M + m
            cb_ = (2 * c + 1) * M + m
            idx_ref[:, ca:ca + 1] = ia
            idx_ref[:, cb_:cb_ + 1] = ib
        q_ref[:, 2 * d * c:2 * d * (c + 1)] = rec
    csum_ref[:, :] += acc.reshape(1, 1)


def kernel(x, codebooks):
    B, T, D = x.shape
    G, M, K, d = codebooks.shape
    N = B * T
    GM = G * M
    P = G // 2
    BLK = min(2048, N)
    f32 = jnp.float32
    xf = x.reshape(N, D)

    cbT = jnp.swapaxes(codebooks, -1, -2)
    zT = jnp.zeros_like(cbT[0::2])
    wd = jnp.concatenate(
        [jnp.concatenate([cbT[0::2], zT], axis=-1),
         jnp.concatenate([zT, cbT[1::2]], axis=-1)],
        axis=-2).astype(jnp.bfloat16)

    mask = jnp.int32(-65536)
    bits = jax.lax.bitcast_convert_type(codebooks, jnp.int32)
    e_hi = jax.lax.bitcast_convert_type(bits & mask, f32)
    rem1 = codebooks - e_hi
    bits1 = jax.lax.bitcast_convert_type(rem1, jnp.int32)
    e_mid = jax.lax.bitcast_convert_type(bits1 & mask, f32)
    e_lo = rem1 - e_mid
    wg = jnp.concatenate([e_hi, e_mid, e_lo],
                         axis=-1).astype(jnp.bfloat16)

    e2 = jnp.sum(codebooks ** 2, axis=-1)
    e2p = jnp.concatenate([e2[0::2], e2[1::2]],
                          axis=-1).reshape(P * M, 2 * K)

    grid = (N // BLK,)
    quant, idx, csum = pl.pallas_call(
        functools.partial(_vq_pair_kernel, G=G, M=M, K=K, d=d),
        grid=grid,
        in_specs=[
            pl.BlockSpec((BLK, D), lambda i: (i, 0)),
            pl.BlockSpec((P, M, 2 * d, 2 * K), lambda i: (0, 0, 0, 0)),
            pl.BlockSpec((G, M, K, 3 * d), lambda i: (0, 0, 0, 0)),
            pl.BlockSpec((P * M, 2 * K), lambda i: (0, 0)),
        ],
        out_specs=[
            pl.BlockSpec((BLK, D), lambda i: (i, 0)),
            pl.BlockSpec((BLK, GM), lambda i: (i, 0)),
            pl.BlockSpec((1, 1), lambda i: (0, 0)),
        ],
        out_shape=[
            jax.ShapeDtypeStruct((N, D), f32),
            jax.ShapeDtypeStruct((N, GM), jnp.int32),
            jax.ShapeDtypeStruct((1, 1), f32),
        ],
        compiler_params=pltpu.CompilerParams(
            dimension_semantics=("arbitrary",)),
    )(xf, wd, wg, e2p)

    quantized = quant.reshape(B, T, D)
    indices = idx.reshape(B, T, GM)
    commit = csum[0, 0] * (_COMMIT / (N * d))
    return quantized, indices, commit


# speedup vs baseline: 1.1136x; 1.1136x over previous
import functools

import jax
import jax.numpy as jnp
from jax.experimental import pallas as pl
from jax.experimental.pallas import tpu as pltpu

_COMMIT = 0.25


def _vq_pair_kernel(x_ref, wd_ref, wg_ref, e2_ref, q_ref, idx_ref, csum_ref,
                    *, G, M, K, d):
    BLK = x_ref.shape[0]
    P = G // 2
    step = pl.program_id(0)

    @pl.when(step == 0)
    def _init():
        csum_ref[:, :] = jnp.zeros((1, 1), jnp.float32)

    acc = jnp.float32(0.0)
    dn = (((1,), (0,)), ((), ()))
    iota = jax.lax.broadcasted_iota(jnp.int32, (BLK, K), 1)
    for c in range(P):
        rp = x_ref[:, 2 * d * c:2 * d * (c + 1)]
        rec = jnp.zeros_like(rp)
        for m in range(M):
            s = rp * rp
            p = s
            for k in range(1, 8):
                p = p + jnp.roll(s, -8 * k, axis=1)
            t = p + jnp.roll(p, -4, axis=1)
            t = t + jnp.roll(t, -2, axis=1)
            t = t + jnp.roll(t, -1, axis=1)
            r2a = t[:, 0:1]
            r2b = t[:, d:d + 1]
            dots = jax.lax.dot_general(
                rp, wd_ref[c, m], dn,
                preferred_element_type=jnp.float32)
            e2row = e2_ref[c * M + m:c * M + m + 1, :]
            da = (r2a - 2.0 * dots[:, :K]) + e2row[:, :K]
            db = (r2b - 2.0 * dots[:, K:]) + e2row[:, K:]
            mina = jnp.min(da, axis=1, keepdims=True)
            ia = jnp.min(jnp.where(da == mina, iota, K), axis=1,
                         keepdims=True)
            minb = jnp.min(db, axis=1, keepdims=True)
            ib = jnp.min(jnp.where(db == minb, iota, K), axis=1,
                         keepdims=True)
            oh = jnp.concatenate(
                [(iota == ia).astype(jnp.float32),
                 (iota == ib).astype(jnp.float32)], axis=1)
            q3 = jax.lax.dot_general(
                oh, wg_ref[c, m], dn,
                preferred_element_type=jnp.float32)
            qp = (q3[:, 0:2 * d] + q3[:, 2 * d:4 * d]) + q3[:, 4 * d:6 * d]
            acc += jnp.sum((qp - rp) ** 2)
            qst = rp + (qp - rp)
            rec = rec + qst
            rp = rp - qst
            ca = (2 * c) * M + m
            cb_ = (2 * c + 1) * M + m
            idx_ref[:, ca:ca + 1] = ia
            idx_ref[:, cb_:cb_ + 1] = ib
        q_ref[:, 2 * d * c:2 * d * (c + 1)] = rec
    csum_ref[:, :] += acc.reshape(1, 1)


def kernel(x, codebooks):
    B, T, D = x.shape
    G, M, K, d = codebooks.shape
    N = B * T
    GM = G * M
    P = G // 2
    BLK = min(2048, N)
    f32 = jnp.float32
    xf = x.reshape(N, D)

    cbT = jnp.swapaxes(codebooks, -1, -2)
    zT = jnp.zeros_like(cbT[0::2])
    wd = jnp.concatenate(
        [jnp.concatenate([cbT[0::2], zT], axis=-1),
         jnp.concatenate([zT, cbT[1::2]], axis=-1)],
        axis=-2).astype(jnp.bfloat16)

    mask = jnp.int32(-65536)
    bits = jax.lax.bitcast_convert_type(codebooks, jnp.int32)
    e_hi = jax.lax.bitcast_convert_type(bits & mask, f32)
    rem1 = codebooks - e_hi
    bits1 = jax.lax.bitcast_convert_type(rem1, jnp.int32)
    e_mid = jax.lax.bitcast_convert_type(bits1 & mask, f32)
    e_lo = rem1 - e_mid
    tiles = []
    for sp in (e_hi, e_mid, e_lo):
        a, b = sp[0::2], sp[1::2]
        z = jnp.zeros_like(a)
        tiles.append(jnp.concatenate(
            [jnp.concatenate([a, z], axis=-1),
             jnp.concatenate([z, b], axis=-1)], axis=-2))
    wg = jnp.concatenate(tiles, axis=-1).astype(jnp.bfloat16)

    e2 = jnp.sum(codebooks ** 2, axis=-1)
    e2p = jnp.concatenate([e2[0::2], e2[1::2]],
                          axis=-1).reshape(P * M, 2 * K)

    grid = (N // BLK,)
    quant, idx, csum = pl.pallas_call(
        functools.partial(_vq_pair_kernel, G=G, M=M, K=K, d=d),
        grid=grid,
        in_specs=[
            pl.BlockSpec((BLK, D), lambda i: (i, 0)),
            pl.BlockSpec((P, M, 2 * d, 2 * K), lambda i: (0, 0, 0, 0)),
            pl.BlockSpec((P, M, 2 * K, 6 * d), lambda i: (0, 0, 0, 0)),
            pl.BlockSpec((P * M, 2 * K), lambda i: (0, 0)),
        ],
        out_specs=[
            pl.BlockSpec((BLK, D), lambda i: (i, 0)),
            pl.BlockSpec((BLK, GM), lambda i: (i, 0)),
            pl.BlockSpec((1, 1), lambda i: (0, 0)),
        ],
        out_shape=[
            jax.ShapeDtypeStruct((N, D), f32),
            jax.ShapeDtypeStruct((N, GM), jnp.int32),
            jax.ShapeDtypeStruct((1, 1), f32),
        ],
        compiler_params=pltpu.CompilerParams(
            dimension_semantics=("arbitrary",)),
    )(xf, wd, wg, e2p)

    quantized = quant.reshape(B, T, D)
    indices = idx.reshape(B, T, GM)
    commit = csum[0, 0] * (_COMMIT / (N * d))
    return quantized, indices, commit
